# Initial kernel scaffold; baseline (speedup 1.0000x reference)
import jax
import jax.numpy as jnp
from jax.experimental import pallas as pl


def kernel(positions, scales, rotations, colors):
    raise NotImplementedError("write your pallas kernel here")


# speedup vs baseline: 45.0805x; 45.0805x over previous
import jax
import jax.numpy as jnp
from jax.experimental import pallas as pl

_N = 4096
_BR = 256
_W = 0.1


def _tc_body(pos_blk, pos_all, col_blk, col_all, scl_blk, rot_blk, out_ref):
    step = pl.program_id(0)
    x = pos_blk[...]
    xa = pos_all[...]
    sqi = jnp.sum(x * x, axis=1, keepdims=True)
    sqj = jnp.sum(xa * xa, axis=1)[None, :]
    xy = jax.lax.dot_general(x, xa, (((1,), (1,)), ((), ())),
                             preferred_element_type=jnp.float32)
    d2 = sqi + sqj - 2.0 * xy
    d = jnp.sqrt(jnp.maximum(d2, 1e-12))
    iota = jax.lax.broadcasted_iota(jnp.int32, (_BR, _N), 1)
    rows = step * _BR + jax.lax.broadcasted_iota(jnp.int32, (_BR, _N), 0)
    big = jnp.float32(jnp.inf)
    d = jnp.where(iota == rows, big, d)
    ci = col_blk[...]
    ca = col_all[...]
    sm = jnp.zeros((_BR,), jnp.float32)
    e2 = jnp.zeros((_BR,), jnp.float32)
    for it in range(5):
        m = jnp.min(d, axis=1)
        am = jnp.min(jnp.where(d == m[:, None], iota, _N), axis=1)
        sel = iota == am[:, None]
        oh = sel.astype(jnp.float32)
        cnb = jax.lax.dot_general(oh, ca, (((1,), (0,)), ((), ())),
                                  preferred_element_type=jnp.float32)
        sm = sm + jnp.sum(jnp.abs(ci - cnb), axis=1)
        if it == 1:
            e2 = m
        d = jnp.where(sel, big, d)
    pos_part = jnp.sum(jnp.exp(-e2))
    smooth_part = jnp.sum(sm) / 15.0
    s = scl_blk[...]
    scale_part = jnp.sum(jnp.abs(s - 1.0)) / 3.0
    mu = jnp.mean(s, axis=1, keepdims=True)
    var_part = jnp.sum((s - mu) ** 2) / 2.0
    q = rot_blk[...]
    qn = jnp.sqrt(jnp.sum(q * q, axis=1))
    rot_part = jnp.sum((qn - 1.0) ** 2)
    col_part = jnp.sum((ci - 0.5) ** 2) / 3.0
    total = (_W * pos_part + _W * (scale_part + var_part)
             + _W * rot_part + _W * (col_part + smooth_part)) / _N

    @pl.when(step == 0)
    def _init():
        out_ref[...] = jnp.zeros((1, 1), jnp.float32)

    out_ref[...] += jnp.reshape(total, (1, 1))


def kernel(positions, scales, rotations, colors):
    grid = _N // _BR
    out = pl.pallas_call(
        _tc_body,
        grid=(grid,),
        in_specs=[
            pl.BlockSpec((_BR, 3), lambda i: (i, 0)),
            pl.BlockSpec((_N, 3), lambda i: (0, 0)),
            pl.BlockSpec((_BR, 3), lambda i: (i, 0)),
            pl.BlockSpec((_N, 3), lambda i: (0, 0)),
            pl.BlockSpec((_BR, 3), lambda i: (i, 0)),
            pl.BlockSpec((_BR, 4), lambda i: (i, 0)),
        ],
        out_specs=pl.BlockSpec((1, 1), lambda i: (0, 0)),
        out_shape=jax.ShapeDtypeStruct((1, 1), jnp.float32),
    )(positions, positions, colors, colors, scales, rotations)
    return out[0, 0]
